# Initial kernel scaffold; baseline (speedup 1.0000x reference)
import jax
import jax.numpy as jnp
from jax.experimental import pallas as pl


def kernel(x, gate_w, W1, b1, W2, b2):
    raise NotImplementedError("write your pallas kernel here")


# speedup vs baseline: 1.6341x; 1.6341x over previous
import functools

import jax
import jax.numpy as jnp
from jax.experimental import pallas as pl
from jax.experimental.pallas import tpu as pltpu

_E = 16
_D_IN = 768
_D_HID = 1536
_D_OUT = 768


def _moe_step(x_ref, gw_ref, w1_ref, b1_ref, w2_ref, b2_ref, out_ref,
              widx_ref, wcol_ref):
    e = pl.program_id(0)
    xf = x_ref[...]

    @pl.when(e == 0)
    def _router():
        logits = jax.lax.dot_general(
            xf, gw_ref[...], (((1,), (1,)), ((), ())),
            preferred_element_type=jnp.float32)
        m = jnp.max(logits, axis=1, keepdims=True)
        lane = jax.lax.broadcasted_iota(jnp.int32, logits.shape, 1)
        idx = jnp.min(jnp.where(logits == m, lane, _E),
                      axis=1, keepdims=True).astype(jnp.float32)
        s = jnp.sum(jnp.exp(logits - m), axis=1, keepdims=True)
        widx_ref[...] = idx
        wcol_ref[...] = 1.0 / (1.0 + 1e-8 * s)

    h = jax.lax.dot_general(
        xf, w1_ref[0], (((1,), (1,)), ((), ())),
        preferred_element_type=jnp.float32)
    h = jnp.maximum(h + b1_ref[0], 0.0)
    y = jax.lax.dot_general(
        h, w2_ref[0], (((1,), (1,)), ((), ())),
        preferred_element_type=jnp.float32)
    y = y + b2_ref[0]

    gate = jnp.where(widx_ref[...] == jnp.float32(1) * e, wcol_ref[...], 0.0)
    contrib = gate * y

    @pl.when(e == 0)
    def _init():
        out_ref[...] = xf + contrib

    @pl.when(e != 0)
    def _acc():
        out_ref[...] += contrib


@functools.partial(jax.jit, static_argnames=("interpret",))
def kernel(x, gate_w, W1, b1, W2, b2, interpret=False):
    orig_shape = x.shape
    xf = x.reshape(-1, orig_shape[-1])
    t = xf.shape[0]

    out = pl.pallas_call(
        _moe_step,
        grid=(_E,),
        in_specs=[
            pl.BlockSpec((t, _D_IN), lambda e: (0, 0)),
            pl.BlockSpec((_E, _D_IN), lambda e: (0, 0)),
            pl.BlockSpec((1, _D_HID, _D_IN), lambda e: (e, 0, 0)),
            pl.BlockSpec((1, 1, _D_HID), lambda e: (e, 0, 0)),
            pl.BlockSpec((1, _D_OUT, _D_HID), lambda e: (e, 0, 0)),
            pl.BlockSpec((1, 1, _D_OUT), lambda e: (e, 0, 0)),
        ],
        out_specs=pl.BlockSpec((t, _D_OUT), lambda e: (0, 0)),
        out_shape=jax.ShapeDtypeStruct((t, _D_OUT), jnp.float32),
        scratch_shapes=[
            pltpu.VMEM((t, 1), jnp.float32),
            pltpu.VMEM((t, 1), jnp.float32),
        ],
        interpret=interpret,
    )(xf, gate_w, W1, b1[:, None, :], W2, b2[:, None, :])

    return out.reshape(orig_shape[:-1] + (_D_OUT,))
